# baseline (speedup 1.0000x reference)
import functools

import jax
import jax.numpy as jnp
from jax import lax
from jax.experimental import pallas as pl
from jax.experimental.pallas import tpu as pltpu
from jax.experimental.pallas import tpu_sc as plsc

_VOCAB = 50000
_TOTAL = 50512
_N = 4096
_EPS = 1e-20
_IGNORE = -100

_NC, _NS = 2, 16
_NW = _NC * _NS
_C = _N // _NW
_L = 16
_P = 32
_W0 = 49920
_WW = _TOTAL - _W0
_WB = _WW // 8

_mesh = plsc.VectorSubcoreMesh(core_axis_name="c", subcore_axis_name="s")


@functools.partial(
    pl.kernel,
    mesh=_mesh,
    compiler_params=pltpu.CompilerParams(needs_layout_passes=False),
    out_type=jax.ShapeDtypeStruct((_N,), jnp.float32),
    scratch_types=[
        pltpu.VMEM((_C,), jnp.int32),
        pltpu.VMEM((_P, 8, _C), jnp.float32),
        pltpu.VMEM((_P, 8, _C), jnp.float32),
        pltpu.VMEM((_C,), jnp.float32),
        pltpu.SemaphoreType.DMA,
    ],
)
def _gather_vocab(scores_t_hbm, target_hbm, out_hbm,
                  tgt_v, ch_a, ch_b, vp_v, sem):
    wid = lax.axis_index("s") * _NC + lax.axis_index("c")
    base = pl.multiple_of(wid * _C, _C)
    pltpu.sync_copy(target_hbm.at[pl.ds(base, _C)], tgt_v)

    bufs = (ch_a, ch_b)

    def fire(p):
        ch = bufs[p % 2]
        copies = []
        for j in range(_P // _L):
            tvec = (tgt_v[pl.ds(p * _P + j * _L, _L)] >> 3) << 3
            for i in range(_L):
                r = j * _L + i
                t0 = pl.multiple_of(tvec[i], 8)
                copies.append(pltpu.async_copy(
                    scores_t_hbm.at[pl.ds(t0, 8), pl.ds(base, _C)],
                    ch.at[r], sem))
        return copies

    def select(p):
        ch = bufs[p % 2]
        for j in range(_P // _L):
            rl = lax.iota(jnp.int32, _L) + j * _L
            sl = pl.ds(p * _P + j * _L, _L)
            t = tgt_v[sl]
            rr = rl + p * _P
            vp_v[sl] = plsc.load_gather(ch, [rl, t & 7, rr])

    nphase = _C // _P
    pending = fire(0)
    for p in range(nphase):
        nxt = fire(p + 1) if p + 1 < nphase else []
        for cp in pending:
            cp.wait()
        select(p)
        pending = nxt
    pltpu.sync_copy(vp_v, out_hbm.at[pl.ds(base, _C)])


def _combine_body(win_ref, aln_ref, tgt_ref, vp_ref, out_ref):
    i = pl.program_id(0)
    ids = jax.lax.broadcasted_iota(jnp.int32, (8, _N), 0) + (_W0 + i * 8)
    want = _VOCAB + aln_ref[...]
    hit = ids == want
    contrib = jnp.sum(jnp.where(hit, win_ref[...], 0.0), axis=0, keepdims=True)

    @pl.when(i == 0)
    def _():
        out_ref[...] = jnp.zeros_like(out_ref)

    total = out_ref[...] + contrib
    out_ref[...] = total

    @pl.when(i == _WB - 1)
    def _():
        a = aln_ref[...]
        t = tgt_ref[...]
        c = jnp.where(a == 0, 0.0, total) + _EPS
        non_copy = (a == 0) | (t != 0)
        probs = jnp.where(non_copy, c + vp_ref[...], c)
        loss = -jnp.log(probs)
        out_ref[...] = jnp.where(t == _IGNORE, 0.0, loss)


def kernel(scores, align, target):
    scores_t = scores.T
    vp = _gather_vocab(scores_t, target)
    loss = pl.pallas_call(
        _combine_body,
        grid=(_WB,),
        in_specs=[
            pl.BlockSpec((8, _N), lambda i: (_W0 // 8 + i, 0)),
            pl.BlockSpec((1, _N), lambda i: (0, 0)),
            pl.BlockSpec((1, _N), lambda i: (0, 0)),
            pl.BlockSpec((1, _N), lambda i: (0, 0)),
        ],
        out_specs=pl.BlockSpec((1, _N), lambda i: (0, 0)),
        out_shape=jax.ShapeDtypeStruct((1, _N), jnp.float32),
    )(scores_t, align.reshape(1, _N), target.reshape(1, _N),
      vp.reshape(1, _N))
    return loss.reshape(_N)


# speedup vs baseline: 1.8518x; 1.8518x over previous
import functools

import jax
import jax.numpy as jnp
from jax import lax
from jax.experimental import pallas as pl
from jax.experimental.pallas import tpu as pltpu
from jax.experimental.pallas import tpu_sc as plsc

_VOCAB = 50000
_TOTAL = 50512
_N = 4096
_EPS = 1e-20
_IGNORE = -100

_NC, _NS = 2, 16
_NW = _NC * _NS
_C = _N // _NW
_L = 16
_P = 32
_WH = 80
_WB = 7

_mesh = plsc.VectorSubcoreMesh(core_axis_name="c", subcore_axis_name="s")


@functools.partial(
    pl.kernel,
    mesh=_mesh,
    compiler_params=pltpu.CompilerParams(needs_layout_passes=False),
    out_type=jax.ShapeDtypeStruct((_N,), jnp.float32),
    scratch_types=[
        pltpu.VMEM((_C,), jnp.int32),
        pltpu.VMEM((_P, 8, _C), jnp.float32),
        pltpu.VMEM((_P, 8, _C), jnp.float32),
        pltpu.VMEM((_C,), jnp.float32),
        pltpu.SemaphoreType.DMA,
    ],
)
def _gather_vocab(scores_t_hbm, target_hbm, out_hbm,
                  tgt_v, ch_a, ch_b, vp_v, sem):
    wid = lax.axis_index("s") * _NC + lax.axis_index("c")
    base = pl.multiple_of(wid * _C, _C)
    pltpu.sync_copy(target_hbm.at[pl.ds(base, _C)], tgt_v)

    bufs = (ch_a, ch_b)

    def fire(p):
        ch = bufs[p % 2]
        copies = []
        for j in range(_P // _L):
            tvec = (tgt_v[pl.ds(p * _P + j * _L, _L)] >> 3) << 3
            for i in range(_L):
                r = j * _L + i
                t0 = pl.multiple_of(tvec[i], 8)
                copies.append(pltpu.async_copy(
                    scores_t_hbm.at[pl.ds(t0, 8), pl.ds(base, _C)],
                    ch.at[r], sem))
        return copies

    def select(p):
        ch = bufs[p % 2]
        for j in range(_P // _L):
            rl = lax.iota(jnp.int32, _L) + j * _L
            sl = pl.ds(p * _P + j * _L, _L)
            t = tgt_v[sl]
            rr = rl + p * _P
            vp_v[sl] = plsc.load_gather(ch, [rl, t & 7, rr])

    nphase = _C // _P
    pending = fire(0)
    for p in range(nphase):
        nxt = fire(p + 1) if p + 1 < nphase else []
        for cp in pending:
            cp.wait()
        select(p)
        pending = nxt
    pltpu.sync_copy(vp_v, out_hbm.at[pl.ds(base, _C)])


def _combine_body(win_ref, aln_ref, tgt_ref, vp_ref, out_ref):
    i = pl.program_id(0)
    off = jax.lax.broadcasted_iota(jnp.int32, (_WH, _N), 0) + i * _WH
    hit = off == aln_ref[...]
    contrib = jnp.sum(jnp.where(hit, win_ref[...], 0.0), axis=0, keepdims=True)

    @pl.when(i == 0)
    def _():
        out_ref[...] = jnp.zeros_like(out_ref)

    total = out_ref[...] + contrib
    out_ref[...] = total

    @pl.when(i == _WB - 1)
    def _():
        a = aln_ref[...]
        t = tgt_ref[...]
        c = jnp.where(a == 0, 0.0, total) + _EPS
        non_copy = (a == 0) | (t != 0)
        probs = jnp.where(non_copy, c + vp_ref[...], c)
        loss = -jnp.log(probs)
        out_ref[...] = jnp.where(t == _IGNORE, 0.0, loss)


def kernel(scores, align, target):
    scores_t = scores.T
    vp = _gather_vocab(scores_t, target)
    loss = pl.pallas_call(
        _combine_body,
        grid=(_WB,),
        in_specs=[
            pl.BlockSpec((_WH, _N), lambda i: (_VOCAB // _WH + i, 0)),
            pl.BlockSpec((1, _N), lambda i: (0, 0)),
            pl.BlockSpec((1, _N), lambda i: (0, 0)),
            pl.BlockSpec((1, _N), lambda i: (0, 0)),
        ],
        out_specs=pl.BlockSpec((1, _N), lambda i: (0, 0)),
        out_shape=jax.ShapeDtypeStruct((1, _N), jnp.float32),
    )(scores_t, align.reshape(1, _N), target.reshape(1, _N),
      vp.reshape(1, _N))
    return loss.reshape(_N)


# speedup vs baseline: 2.5112x; 1.3561x over previous
import functools

import jax
import jax.numpy as jnp
from jax import lax
from jax.experimental import pallas as pl
from jax.experimental.pallas import tpu as pltpu
from jax.experimental.pallas import tpu_sc as plsc

_VOCAB = 50000
_TOTAL = 50512
_N = 4096
_EPS = 1e-20
_IGNORE = -100

_NC, _NS = 2, 16
_NW = _NC * _NS
_C = _N // _NW
_L = 16

_mesh = plsc.VectorSubcoreMesh(core_axis_name="c", subcore_axis_name="s")


@functools.partial(
    pl.kernel,
    mesh=_mesh,
    compiler_params=pltpu.CompilerParams(needs_layout_passes=False),
    out_type=jax.ShapeDtypeStruct((_N,), jnp.float32),
    scratch_types=[
        pltpu.VMEM((_C,), jnp.int32),
        pltpu.VMEM((_C,), jnp.int32),
        pltpu.VMEM((_C,), jnp.int32),
        pltpu.VMEM((_C, _C), jnp.float32),
        pltpu.VMEM((_C, _C), jnp.float32),
        pltpu.VMEM((_C,), jnp.float32),
        pltpu.SemaphoreType.DMA,
        pltpu.SemaphoreType.DMA,
    ],
)
def _gather_probs(scores_t_hbm, align_hbm, target_hbm, out_hbm,
                  tgt_v, aln_v, ci_v, ch_t, ch_c, o_v, sem, sem2):
    wid = lax.axis_index("s") * _NC + lax.axis_index("c")
    base = pl.multiple_of(wid * _C, _C)
    pltpu.sync_copy(target_hbm.at[pl.ds(base, _C)], tgt_v)
    pltpu.sync_copy(align_hbm.at[pl.ds(base, _C)], aln_v)
    for j in range(_C // _L):
        sl = pl.ds(j * _L, _L)
        ci_v[sl] = aln_v[sl] + _VOCAB
    g1 = pltpu.async_copy(scores_t_hbm.at[tgt_v, pl.ds(base, _C)], ch_t, sem)
    g2 = pltpu.async_copy(scores_t_hbm.at[ci_v, pl.ds(base, _C)], ch_c, sem2)
    g1.wait()
    g2.wait()
    for j in range(_C // _L):
        sl = pl.ds(j * _L, _L)
        r = lax.iota(jnp.int32, _L) + j * _L
        t = tgt_v[sl]
        a = aln_v[sl]
        v = plsc.load_gather(ch_t, [r, r])
        c = plsc.load_gather(ch_c, [r, r])
        c = jnp.where(a == 0, 0.0, c) + _EPS
        non_copy = (a == 0) | (t != 0)
        o_v[sl] = jnp.where(non_copy, c + v, c)
    pltpu.sync_copy(o_v, out_hbm.at[pl.ds(base, _C)])


def _loss_body(p_ref, t_ref, o_ref):
    loss = -jnp.log(p_ref[...])
    o_ref[...] = jnp.where(t_ref[...] == _IGNORE, 0.0, loss)


def kernel(scores, align, target):
    probs = _gather_probs(scores.T, align, target)
    loss = pl.pallas_call(
        _loss_body,
        out_shape=jax.ShapeDtypeStruct((_N // 128, 128), jnp.float32),
    )(probs.reshape(_N // 128, 128), target.reshape(_N // 128, 128))
    return loss.reshape(_N)


# speedup vs baseline: 2.6489x; 1.0548x over previous
import functools

import jax
import jax.numpy as jnp
from jax import lax
from jax.experimental import pallas as pl
from jax.experimental.pallas import tpu as pltpu
from jax.experimental.pallas import tpu_sc as plsc

_VOCAB = 50000
_TOTAL = 50512
_N = 4096
_EPS = 1e-20
_IGNORE = -100
_LN2 = 0.6931471805599453

_NC, _NS = 2, 16
_NW = _NC * _NS
_C = _N // _NW
_L = 16

_mesh = plsc.VectorSubcoreMesh(core_axis_name="c", subcore_axis_name="s")


def _neg_log(p):
    bits = plsc.bitcast(p, jnp.int32)
    e = (bits >> 23) - 127
    m = plsc.bitcast((bits & 0x007FFFFF) | 0x3F800000, jnp.float32)
    s = (m - 1.0) / (m + 1.0)
    s2 = s * s
    poly = 1.0 + s2 * (1.0 / 3.0 + s2 * (1.0 / 5.0 + s2 * (1.0 / 7.0 + s2 * (1.0 / 9.0))))
    return -(e.astype(jnp.float32) * _LN2 + 2.0 * s * poly)


@functools.partial(
    pl.kernel,
    mesh=_mesh,
    compiler_params=pltpu.CompilerParams(needs_layout_passes=False),
    out_type=jax.ShapeDtypeStruct((_N,), jnp.float32),
    scratch_types=[
        pltpu.VMEM((_C,), jnp.int32),
        pltpu.VMEM((_C,), jnp.int32),
        pltpu.VMEM((_C,), jnp.int32),
        pltpu.VMEM((_C, _C), jnp.float32),
        pltpu.VMEM((_C, _C), jnp.float32),
        pltpu.VMEM((_C,), jnp.float32),
        pltpu.SemaphoreType.DMA,
        pltpu.SemaphoreType.DMA,
    ],
)
def _loss_kernel(scores_t_hbm, align_hbm, target_hbm, out_hbm,
                 tgt_v, aln_v, ci_v, ch_t, ch_c, o_v, sem, sem2):
    wid = lax.axis_index("s") * _NC + lax.axis_index("c")
    base = pl.multiple_of(wid * _C, _C)
    pltpu.sync_copy(target_hbm.at[pl.ds(base, _C)], tgt_v)
    g1 = pltpu.async_copy(scores_t_hbm.at[tgt_v, pl.ds(base, _C)], ch_t, sem)
    pltpu.sync_copy(align_hbm.at[pl.ds(base, _C)], aln_v)
    for j in range(_C // _L):
        sl = pl.ds(j * _L, _L)
        ci_v[sl] = aln_v[sl] + _VOCAB
    g2 = pltpu.async_copy(scores_t_hbm.at[ci_v, pl.ds(base, _C)], ch_c, sem2)
    g1.wait()
    g2.wait()
    for j in range(_C // _L):
        sl = pl.ds(j * _L, _L)
        r = lax.iota(jnp.int32, _L) + j * _L
        t = tgt_v[sl]
        a = aln_v[sl]
        v = plsc.load_gather(ch_t, [r, r])
        c = plsc.load_gather(ch_c, [r, r])
        c = jnp.where(a == 0, 0.0, c) + _EPS
        non_copy = (a == 0) | (t != 0)
        probs = jnp.where(non_copy, c + v, c)
        loss = _neg_log(probs)
        o_v[sl] = jnp.where(t == _IGNORE, 0.0, loss)
    pltpu.sync_copy(o_v, out_hbm.at[pl.ds(base, _C)])


def kernel(scores, align, target):
    return _loss_kernel(scores.T, align, target)


# speedup vs baseline: 2.6561x; 1.0027x over previous
import functools

import jax
import jax.numpy as jnp
from jax import lax
from jax.experimental import pallas as pl
from jax.experimental.pallas import tpu as pltpu
from jax.experimental.pallas import tpu_sc as plsc

_VOCAB = 50000
_TOTAL = 50512
_N = 4096
_EPS = 1e-20
_IGNORE = -100
_LN2 = 0.6931471805599453

_NC, _NS = 2, 16
_NW = _NC * _NS
_C = _N // _NW
_L = 16

_mesh = plsc.VectorSubcoreMesh(core_axis_name="c", subcore_axis_name="s")


def _neg_log(p):
    bits = plsc.bitcast(p, jnp.int32)
    e = (bits >> 23) - 127
    m = plsc.bitcast((bits & 0x007FFFFF) | 0x3F800000, jnp.float32)
    s = (m - 1.0) / (m + 1.0)
    s2 = s * s
    poly = 1.0 + s2 * (1.0 / 3.0 + s2 * (1.0 / 5.0 + s2 * (1.0 / 7.0 + s2 * (1.0 / 9.0))))
    return -(e.astype(jnp.float32) * _LN2 + 2.0 * s * poly)


@functools.partial(
    pl.kernel,
    mesh=_mesh,
    compiler_params=pltpu.CompilerParams(needs_layout_passes=False),
    out_type=jax.ShapeDtypeStruct((_N,), jnp.float32),
    scratch_types=[
        pltpu.VMEM((_C,), jnp.int32),
        pltpu.VMEM((_C,), jnp.int32),
        pltpu.VMEM((_C,), jnp.int32),
        pltpu.VMEM((_C, _C), jnp.float32),
        pltpu.VMEM((_C, _C), jnp.float32),
        pltpu.VMEM((_C,), jnp.float32),
        pltpu.SemaphoreType.DMA,
        pltpu.SemaphoreType.DMA,
        pltpu.SemaphoreType.DMA,
        pltpu.SemaphoreType.DMA,
    ],
)
def _loss_kernel(scores_t_hbm, align_hbm, target_hbm, out_hbm,
                 tgt_v, aln_v, ci_v, ch_t, ch_c, o_v, sem, sem2, sem3, sem4):
    wid = lax.axis_index("s") * _NC + lax.axis_index("c")
    base = pl.multiple_of(wid * _C, _C)
    _H = _C // 2
    pltpu.sync_copy(target_hbm.at[pl.ds(base, _C)], tgt_v)
    g1 = pltpu.async_copy(
        scores_t_hbm.at[tgt_v.at[pl.ds(0, _H)], pl.ds(base, _C)],
        ch_t.at[pl.ds(0, _H)], sem)
    g3 = pltpu.async_copy(
        scores_t_hbm.at[tgt_v.at[pl.ds(_H, _H)], pl.ds(base, _C)],
        ch_t.at[pl.ds(_H, _H)], sem3)
    pltpu.sync_copy(align_hbm.at[pl.ds(base, _C)], aln_v)
    for j in range(_C // _L):
        sl = pl.ds(j * _L, _L)
        ci_v[sl] = aln_v[sl] + _VOCAB
    g2 = pltpu.async_copy(
        scores_t_hbm.at[ci_v.at[pl.ds(0, _H)], pl.ds(base, _C)],
        ch_c.at[pl.ds(0, _H)], sem2)
    g4 = pltpu.async_copy(
        scores_t_hbm.at[ci_v.at[pl.ds(_H, _H)], pl.ds(base, _C)],
        ch_c.at[pl.ds(_H, _H)], sem4)
    g1.wait()
    g2.wait()
    g3.wait()
    g4.wait()
    for j in range(_C // _L):
        sl = pl.ds(j * _L, _L)
        r = lax.iota(jnp.int32, _L) + j * _L
        t = tgt_v[sl]
        a = aln_v[sl]
        v = plsc.load_gather(ch_t, [r, r])
        c = plsc.load_gather(ch_c, [r, r])
        c = jnp.where(a == 0, 0.0, c) + _EPS
        non_copy = (a == 0) | (t != 0)
        probs = jnp.where(non_copy, c + v, c)
        loss = _neg_log(probs)
        o_v[sl] = jnp.where(t == _IGNORE, 0.0, loss)
    pltpu.sync_copy(o_v, out_hbm.at[pl.ds(base, _C)])


def kernel(scores, align, target):
    return _loss_kernel(scores.T, align, target)
